# baseline (speedup 1.0000x reference)
import functools

import jax
import jax.numpy as jnp
from jax import lax
from jax.experimental import pallas as pl
from jax.experimental.pallas import tpu as pltpu
from jax.experimental.pallas import tpu_sc as plsc

_LANES = 16


def kernel(input_ids, segment_ids, token_table, seg_table, pos_table):
    B, L = input_ids.shape
    V, D = token_table.shape

    posseg = pos_table[:L] + seg_table[0][None, :]

    NC, NS = 2, 16
    NW = NC * NS
    bpw = B // NW

    mesh = plsc.VectorSubcoreMesh(core_axis_name="c", subcore_axis_name="s")

    @functools.partial(
        pl.kernel,
        mesh=mesh,
        out_type=jax.ShapeDtypeStruct((B, L, D), jnp.float32),
        scratch_types=[
            pltpu.VMEM((L,), jnp.int32),
            pltpu.VMEM((L, D), jnp.float32),
            pltpu.VMEM((L, D), jnp.float32),
            pltpu.SemaphoreType.DMA,
        ],
        compiler_params=pltpu.CompilerParams(use_tc_tiling_on_sc=False),
    )
    def emb_kernel(ids_hbm, posseg_hbm, tok_hbm, out_hbm, idx_v, rows_v, ps_v, sem):
        wid = lax.axis_index("s") * NC + lax.axis_index("c")
        pltpu.sync_copy(posseg_hbm, ps_v)

        def chunk(c, carry):
            b = wid * bpw + c
            pltpu.sync_copy(ids_hbm.at[b], idx_v)
            g1 = pltpu.async_copy(
                tok_hbm.at[idx_v.at[pl.ds(0, 128)]], rows_v.at[pl.ds(0, 128)], sem
            )
            g2 = pltpu.async_copy(
                tok_hbm.at[idx_v.at[pl.ds(128, L - 128)]],
                rows_v.at[pl.ds(128, L - 128)],
                sem,
            )
            g1.wait()
            g2.wait()

            def addrow(r, carry2):
                for j in range(D // _LANES):
                    sl = pl.ds(j * _LANES, _LANES)
                    rows_v[r, sl] = rows_v[r, sl] + ps_v[r, sl]
                return carry2

            lax.fori_loop(0, L, addrow, 0)
            pltpu.sync_copy(rows_v, out_hbm.at[b])
            return carry

        lax.fori_loop(0, bpw, chunk, 0)

    return emb_kernel(input_ids, posseg, token_table)
